# baseline (speedup 1.0000x reference)
import functools

import jax
import jax.numpy as jnp
from jax import lax
from jax.experimental import pallas as pl
from jax.experimental.pallas import tpu as pltpu
from jax.experimental.pallas import tpu_sc as plsc

N = 10000
D = 128
DH = D // 2
E = 320000
EA = E + N

NC = 2
NS = 16
NW = NC * NS
K = 128
NBUF = 4
C = 84
EPAD = NW * C * K
NPAD = 10240
RPT = NPAD // NS

_mesh = plsc.VectorSubcoreMesh(
    core_axis_name="c", subcore_axis_name="s", num_cores=NC, num_subcores=NS
)


@functools.partial(
    pl.kernel,
    out_type=jax.ShapeDtypeStruct((NC, NPAD), jnp.float32),
    mesh=_mesh,
    scratch_types=[
        pltpu.VMEM((C, K), jnp.int32),
        pltpu.VMEM((C, K), jnp.float32),
        pltpu.VMEM((RPT,), jnp.float32),
        pltpu.VMEM_SHARED((NPAD,), jnp.float32),
    ],
)
def _sc_deg(dst_hbm, w_hbm, out_hbm, dst_v, w_v, buf_v, deg_s):
    cid = lax.axis_index("c")
    sid = lax.axis_index("s")
    wid = sid * NC + cid

    def _zero(i, _):
        buf_v[pl.ds(i * 16, 16)] = jnp.zeros((16,), jnp.float32)
        return 0

    lax.fori_loop(0, RPT // 16, _zero, 0)
    pltpu.sync_copy(buf_v, deg_s.at[pl.ds(sid * RPT, RPT)])
    plsc.subcore_barrier()

    pltpu.sync_copy(dst_hbm.at[wid], dst_v)
    pltpu.sync_copy(w_hbm.at[wid], w_v)

    def _body(j, _):
        pltpu.sync_copy(w_v.at[j], deg_s.at[dst_v.at[j]], add=True)
        return 0

    lax.fori_loop(0, C, _body, 0)
    plsc.subcore_barrier()

    pltpu.sync_copy(deg_s.at[pl.ds(sid * RPT, RPT)], buf_v)
    pltpu.sync_copy(buf_v, out_hbm.at[cid, pl.ds(sid * RPT, RPT)])


@functools.partial(
    pl.kernel,
    out_type=jax.ShapeDtypeStruct((NW, C, K), jnp.float32),
    mesh=_mesh,
    scratch_types=[
        pltpu.VMEM((C, K), jnp.int32),
        pltpu.VMEM((C, K), jnp.int32),
        pltpu.VMEM((C, K), jnp.float32),
        pltpu.VMEM((K,), jnp.float32),
        pltpu.VMEM((K,), jnp.float32),
        pltpu.SemaphoreType.DMA,
        pltpu.SemaphoreType.DMA,
    ],
)
def _sc_norm(dinv_hbm, src_hbm, dst_hbm, w_hbm, out_hbm,
             src_v, dst_v, w_v, a_v, b_v, sema, semb):
    cid = lax.axis_index("c")
    sid = lax.axis_index("s")
    wid = sid * NC + cid

    pltpu.sync_copy(src_hbm.at[wid], src_v)
    pltpu.sync_copy(dst_hbm.at[wid], dst_v)
    pltpu.sync_copy(w_hbm.at[wid], w_v)

    def _body(j, _):
        da = pltpu.async_copy(dinv_hbm.at[src_v.at[j]], a_v, sema)
        db = pltpu.async_copy(dinv_hbm.at[dst_v.at[j]], b_v, semb)
        da.wait()
        db.wait()
        for i in range(K // 16):
            sl = pl.ds(i * 16, 16)
            w_v[j, sl] = w_v[j, sl] * (a_v[sl] * b_v[sl])
        return 0

    lax.fori_loop(0, C, _body, 0)
    pltpu.sync_copy(w_v, out_hbm.at[wid])


@functools.partial(
    pl.kernel,
    out_type=(
        jax.ShapeDtypeStruct((NC, NPAD, DH), jnp.float32),
        jax.ShapeDtypeStruct((NC, NPAD, DH), jnp.float32),
    ),
    mesh=_mesh,
    compiler_params=pltpu.CompilerParams(use_tc_tiling_on_sc=False),
    scratch_types=[
        pltpu.VMEM((C, K), jnp.int32),
        pltpu.VMEM((C, K), jnp.int32),
        pltpu.VMEM((C, K), jnp.float32),
        [pltpu.VMEM((K, DH), jnp.float32)] * NBUF,
        pltpu.VMEM_SHARED((NPAD, DH), jnp.float32),
        [pltpu.SemaphoreType.DMA] * NBUF,
        [pltpu.SemaphoreType.DMA] * NBUF,
    ],
)
def _sc_spmm(g_lo_hbm, g_hi_hbm, src_hbm, dst_hbm, w_hbm, out_lo, out_hi,
             src_v, dst_v, w_v, rows, acc_s, gsems, ssems):
    cid = lax.axis_index("c")
    sid = lax.axis_index("s")
    wid = sid * NC + cid
    r0 = rows[0]

    pltpu.sync_copy(src_hbm.at[wid], src_v)
    pltpu.sync_copy(dst_hbm.at[wid], dst_v)
    pltpu.sync_copy(w_hbm.at[wid], w_v)

    for g_hbm, o_hbm in ((g_lo_hbm, out_lo), (g_hi_hbm, out_hi)):
        def _zero(r, _):
            for i in range(DH // 16):
                r0[r, pl.ds(i * 16, 16)] = jnp.zeros((16,), jnp.float32)
            return 0

        lax.fori_loop(0, K, _zero, 0)
        for t in range(RPT // K):
            pltpu.sync_copy(r0, acc_s.at[pl.ds(sid * RPT + t * K, K)])
        plsc.subcore_barrier()

        for b in range(NBUF):
            pltpu.async_copy(g_hbm.at[src_v.at[b]], rows[b], gsems[b])

        def _piece(p, _):
            sdescs = []
            for b in range(NBUF):
                j = p * NBUF + b
                pltpu.make_async_copy(
                    g_hbm.at[pl.ds(0, K)], rows[b], gsems[b]
                ).wait()

                def _scale(g, _):
                    wvec = w_v[j, pl.ds(g * 16, 16)]
                    for r in range(16):
                        s = wvec[r]
                        for i in range(DH // 16):
                            sl = pl.ds(i * 16, 16)
                            rows[b][g * 16 + r, sl] = rows[b][g * 16 + r, sl] * s
                    return 0

                lax.fori_loop(0, K // 16, _scale, 0)
                sdescs.append(pltpu.async_copy(
                    rows[b], acc_s.at[dst_v.at[j]], ssems[b], add=True
                ))
            for b in range(NBUF):
                jr = p * NBUF + b + NBUF
                sdescs[b].wait()

                @pl.when(jr < C)
                def _():
                    pltpu.async_copy(g_hbm.at[src_v.at[jr]], rows[b], gsems[b])

            return 0

        lax.fori_loop(0, C // NBUF, _piece, 0)
        plsc.subcore_barrier()

        for t in range(RPT // K):
            pltpu.sync_copy(acc_s.at[pl.ds(sid * RPT + t * K, K)], r0)
            pltpu.sync_copy(r0, o_hbm.at[cid, pl.ds(sid * RPT + t * K, K)])
        plsc.subcore_barrier()


def _tc_dinv_body(deg_ref, out_ref):
    deg = deg_ref[0] + deg_ref[1]
    out_ref[...] = jnp.where(deg > 0, lax.rsqrt(jnp.maximum(deg, 1e-12)), 0.0)


def _tc_embed_body(x_ref, we_ref, w1_ref, glo_ref, ghi_ref):
    h0 = jnp.dot(x_ref[...], we_ref[...], preferred_element_type=jnp.float32)
    g = jnp.dot(h0, w1_ref[...], preferred_element_type=jnp.float32)
    glo_ref[...] = g[:, :DH]
    ghi_ref[...] = g[:, DH:]


def _tc_mid_body(plo_ref, phi_ref, b_ref, w1_ref, glo_ref, ghi_ref):
    s = jnp.concatenate(
        [plo_ref[0, :N, :] + plo_ref[1, :N, :],
         phi_ref[0, :N, :] + phi_ref[1, :N, :]], axis=1
    )
    a = jnp.maximum(s + b_ref[...], 0.0)
    g = jnp.dot(a, w1_ref[...], preferred_element_type=jnp.float32)
    glo_ref[...] = g[:, :DH]
    ghi_ref[...] = g[:, DH:]


def _tc_final_body(plo_ref, phi_ref, b_ref, out_ref):
    s = jnp.concatenate(
        [plo_ref[0, :N, :] + plo_ref[1, :N, :],
         phi_ref[0, :N, :] + phi_ref[1, :N, :]], axis=1
    )
    out_ref[...] = s + b_ref[...]


def _tc_call(body, out_shapes, *args):
    outs = pl.pallas_call(
        body,
        out_shape=[jax.ShapeDtypeStruct(s, jnp.float32) for s in out_shapes],
    )(*args)
    return outs[0] if len(out_shapes) == 1 else outs


def kernel(x, pp_edge_index, edge_weight, W_embed, W1, b1):
    src = pp_edge_index[0]
    dst = pp_edge_index[1]

    loop_idx = jnp.arange(N, dtype=jnp.int32)
    pad = EPAD - EA
    spread = (jnp.arange(pad, dtype=jnp.int32) * 97) % N
    src_p = jnp.concatenate([src, loop_idx, spread]).reshape(NW, C, K)
    dst_p = jnp.concatenate([dst, loop_idx, spread]).reshape(NW, C, K)
    w_p = jnp.concatenate(
        [edge_weight, jnp.ones((N,), jnp.float32), jnp.zeros((pad,), jnp.float32)]
    ).reshape(NW, C, K)

    deg_part = _sc_deg(dst_p, w_p)
    dinv = _tc_call(_tc_dinv_body, [(NPAD,)], deg_part)
    wn = _sc_norm(dinv, src_p, dst_p, w_p)

    b_row = b1[None, :]
    g1_lo, g1_hi = _tc_call(_tc_embed_body, [(N, DH), (N, DH)],
                            x, W_embed, W1)
    p1_lo, p1_hi = _sc_spmm(g1_lo, g1_hi, src_p, dst_p, wn)
    g2_lo, g2_hi = _tc_call(_tc_mid_body, [(N, DH), (N, DH)],
                            p1_lo, p1_hi, b_row, W1)
    p2_lo, p2_hi = _sc_spmm(g2_lo, g2_hi, src_p, dst_p, wn)
    out = _tc_call(_tc_final_body, [(N, D)], p2_lo, p2_hi, b_row)
    return out


# speedup vs baseline: 2.7635x; 2.7635x over previous
import functools

import jax
import jax.numpy as jnp
from jax import lax
from jax.experimental import pallas as pl
from jax.experimental.pallas import tpu as pltpu
from jax.experimental.pallas import tpu_sc as plsc

N = 10000
D = 128
E = 320000

NC = 2
NS = 16
NW = NC * NS
K = 128
NBUF = 2
C = 80
EPAD = NW * C * K
NPAD = 10240
RPT = NPAD // NS

_mesh = plsc.VectorSubcoreMesh(
    core_axis_name="c", subcore_axis_name="s", num_cores=NC, num_subcores=NS
)


@functools.partial(
    pl.kernel,
    out_type=jax.ShapeDtypeStruct((NC, NPAD), jnp.float32),
    mesh=_mesh,
    scratch_types=[
        pltpu.VMEM((C, K), jnp.int32),
        pltpu.VMEM((C, K), jnp.float32),
        pltpu.VMEM((RPT,), jnp.float32),
        pltpu.VMEM_SHARED((NPAD,), jnp.float32),
    ],
)
def _sc_deg(dst_hbm, w_hbm, out_hbm, dst_v, w_v, buf_v, deg_s):
    cid = lax.axis_index("c")
    sid = lax.axis_index("s")
    wid = sid * NC + cid

    def _zero(i, _):
        buf_v[pl.ds(i * 16, 16)] = jnp.zeros((16,), jnp.float32)
        return 0

    lax.fori_loop(0, RPT // 16, _zero, 0)
    pltpu.sync_copy(buf_v, deg_s.at[pl.ds(sid * RPT, RPT)])
    plsc.subcore_barrier()

    pltpu.sync_copy(dst_hbm.at[wid], dst_v)
    pltpu.sync_copy(w_hbm.at[wid], w_v)

    def _body(j, _):
        pltpu.sync_copy(w_v.at[j], deg_s.at[dst_v.at[j]], add=True)
        return 0

    lax.fori_loop(0, C, _body, 0)
    plsc.subcore_barrier()

    pltpu.sync_copy(deg_s.at[pl.ds(sid * RPT, RPT)], buf_v)
    pltpu.sync_copy(buf_v, out_hbm.at[cid, pl.ds(sid * RPT, RPT)])


@functools.partial(
    pl.kernel,
    out_type=jax.ShapeDtypeStruct((NC, NPAD, D), jnp.float32),
    mesh=_mesh,
    scratch_types=[
        [pltpu.VMEM((K,), jnp.int32)] * NBUF,
        [pltpu.VMEM((K,), jnp.int32)] * NBUF,
        [pltpu.VMEM((K,), jnp.float32)] * NBUF,
        [pltpu.VMEM((K, D), jnp.float32)] * NBUF,
        pltpu.VMEM_SHARED((NPAD, D), jnp.float32),
        [pltpu.SemaphoreType.DMA] * NBUF,
        [pltpu.SemaphoreType.DMA] * NBUF,
        [pltpu.SemaphoreType.DMA] * NBUF,
        [pltpu.SemaphoreType.DMA] * NBUF,
    ],
)
def _sc_spmm(g_hbm, src_hbm, dst_hbm, w_hbm, out_hbm,
             srcs, dsts, ws, rows, acc_s, gsems, ssems, dsems, wsems):
    cid = lax.axis_index("c")
    sid = lax.axis_index("s")
    wid = sid * NC + cid
    r0 = rows[0]

    def _stage(b, j):
        pltpu.async_copy(src_hbm.at[wid, j], srcs[b], ssems[b])
        pltpu.async_copy(dst_hbm.at[wid, j], dsts[b], dsems[b])
        pltpu.async_copy(w_hbm.at[wid, j], ws[b], wsems[b])

    def _stage_wait(b):
        pltpu.make_async_copy(src_hbm.at[wid, 0], srcs[b], ssems[b]).wait()
        pltpu.make_async_copy(dst_hbm.at[wid, 0], dsts[b], dsems[b]).wait()
        pltpu.make_async_copy(w_hbm.at[wid, 0], ws[b], wsems[b]).wait()

    def _gather_wait(b):
        pltpu.make_async_copy(g_hbm.at[pl.ds(0, K)], rows[b], gsems[b]).wait()

    def _zero(r, _):
        for i in range(D // 16):
            r0[r, pl.ds(i * 16, 16)] = jnp.zeros((16,), jnp.float32)
        return 0

    lax.fori_loop(0, K, _zero, 0)
    for t in range(RPT // K):
        pltpu.sync_copy(r0, acc_s.at[pl.ds(sid * RPT + t * K, K)])
    plsc.subcore_barrier()

    for b in range(NBUF):
        _stage(b, b)
    _stage_wait(0)
    pltpu.async_copy(g_hbm.at[srcs[0]], rows[0], gsems[0])

    def _piece(p, _):
        for b in range(NBUF):
            j = p * NBUF + b
            o = (b + 1) % NBUF
            _stage_wait(o)
            pltpu.async_copy(g_hbm.at[srcs[o]], rows[o], gsems[o])

            _gather_wait(b)

            def _scale(g, _):
                wvec = ws[b][pl.ds(g * 16, 16)]
                for r in range(16):
                    sc = wvec[r]
                    for i in range(D // 16):
                        sl = pl.ds(i * 16, 16)
                        rows[b][g * 16 + r, sl] = rows[b][g * 16 + r, sl] * sc
                return 0

            lax.fori_loop(0, K // 16, _scale, 0)
            pltpu.sync_copy(rows[b], acc_s.at[dsts[b]], add=True)
            _stage(b, jnp.minimum(j + NBUF, C - 1))
        return 0

    lax.fori_loop(0, C // NBUF, _piece, 0)
    _gather_wait(C % NBUF)
    for b in range(1, NBUF):
        _stage_wait(b)
    plsc.subcore_barrier()

    for t in range(RPT // K):
        pltpu.sync_copy(acc_s.at[pl.ds(sid * RPT + t * K, K)], r0)
        pltpu.sync_copy(r0, out_hbm.at[cid, pl.ds(sid * RPT + t * K, K)])


def _tc_dinv_body(deg_ref, out_ref):
    gid = lax.broadcasted_iota(jnp.int32, (NPAD,), 0)
    deg = deg_ref[0] + deg_ref[1] + jnp.where(gid < N, 1.0, 0.0)
    out_ref[...] = jnp.where(deg > 0, lax.rsqrt(jnp.maximum(deg, 1e-12)), 0.0)


def _tc_embed_body(x_ref, we_ref, w1_ref, dinv_ref, g_ref):
    h0 = jnp.dot(x_ref[...], we_ref[...], preferred_element_type=jnp.float32)
    h1 = jnp.dot(h0, w1_ref[...], preferred_element_type=jnp.float32)
    g_ref[...] = dinv_ref[...] * h1


def _tc_step_body(p_ref, g_ref, dinv_ref, b_ref, w1_ref, y_ref, gn_ref):
    s = p_ref[0, :N, :] + p_ref[1, :N, :] + g_ref[...]
    y = dinv_ref[...] * s + b_ref[...]
    y_ref[...] = y
    a = jnp.maximum(y, 0.0)
    h = jnp.dot(a, w1_ref[...], preferred_element_type=jnp.float32)
    gn_ref[...] = dinv_ref[...] * h


def _tc_call(body, out_shapes, *args):
    outs = pl.pallas_call(
        body,
        out_shape=[jax.ShapeDtypeStruct(s, jnp.float32) for s in out_shapes],
    )(*args)
    return outs[0] if len(out_shapes) == 1 else outs


def kernel(x, pp_edge_index, edge_weight, W_embed, W1, b1):
    src = pp_edge_index[0]
    dst = pp_edge_index[1]

    pad = EPAD - E
    spread = (jnp.arange(pad, dtype=jnp.int32) * 97) % N
    src_p = jnp.concatenate([src, spread]).reshape(NW, C, K)
    dst_p = jnp.concatenate([dst, spread]).reshape(NW, C, K)
    w_p = jnp.concatenate(
        [edge_weight, jnp.zeros((pad,), jnp.float32)]
    ).reshape(NW, C, K)

    deg_part = _sc_deg(dst_p, w_p)
    dinv = _tc_call(_tc_dinv_body, [(NPAD,)], deg_part)
    dinv_col = dinv[:N, None]

    b_row = b1[None, :]
    g1 = _tc_call(_tc_embed_body, [(N, D)], x, W_embed, W1, dinv_col)

    p1 = _sc_spmm(g1, src_p, dst_p, w_p)
    _, g2 = _tc_call(_tc_step_body, [(N, D), (N, D)],
                     p1, g1, dinv_col, b_row, W1)
    p2 = _sc_spmm(g2, src_p, dst_p, w_p)
    out, _ = _tc_call(_tc_step_body, [(N, D), (N, D)],
                      p2, g2, dinv_col, b_row, W1)
    return out


# speedup vs baseline: 2.9410x; 1.0642x over previous
import functools

import jax
import jax.numpy as jnp
from jax import lax
from jax.experimental import pallas as pl
from jax.experimental.pallas import tpu as pltpu
from jax.experimental.pallas import tpu_sc as plsc

N = 10000
D = 128
E = 320000

NC = 2
NS = 16
NW = NC * NS
K = 128
NBUF = 2
C = 80
EPAD = NW * C * K
NPAD = 10240
RPT = NPAD // NS

_mesh = plsc.VectorSubcoreMesh(
    core_axis_name="c", subcore_axis_name="s", num_cores=NC, num_subcores=NS
)


@functools.partial(
    pl.kernel,
    out_type=jax.ShapeDtypeStruct((NC, NPAD), jnp.float32),
    mesh=_mesh,
    scratch_types=[
        pltpu.VMEM((C, K), jnp.int32),
        pltpu.VMEM((C, K), jnp.float32),
        pltpu.VMEM((RPT,), jnp.float32),
        pltpu.VMEM_SHARED((NPAD,), jnp.float32),
    ],
)
def _sc_deg(dst_hbm, w_hbm, out_hbm, dst_v, w_v, buf_v, deg_s):
    cid = lax.axis_index("c")
    sid = lax.axis_index("s")
    wid = sid * NC + cid

    def _zero(i, _):
        buf_v[pl.ds(i * 16, 16)] = jnp.zeros((16,), jnp.float32)
        return 0

    lax.fori_loop(0, RPT // 16, _zero, 0)
    pltpu.sync_copy(buf_v, deg_s.at[pl.ds(sid * RPT, RPT)])
    plsc.subcore_barrier()

    pltpu.sync_copy(dst_hbm.at[wid], dst_v)
    pltpu.sync_copy(w_hbm.at[wid], w_v)

    def _body(j, _):
        pltpu.sync_copy(w_v.at[j], deg_s.at[dst_v.at[j]], add=True)
        return 0

    lax.fori_loop(0, C, _body, 0)
    plsc.subcore_barrier()

    pltpu.sync_copy(deg_s.at[pl.ds(sid * RPT, RPT)], buf_v)
    pltpu.sync_copy(buf_v, out_hbm.at[cid, pl.ds(sid * RPT, RPT)])


NE = 4
@functools.partial(
    pl.kernel,
    out_type=jax.ShapeDtypeStruct((NC, NPAD, D), jnp.float32),
    mesh=_mesh,
    scratch_types=[
        [pltpu.VMEM((K,), jnp.int32)] * NE,
        [pltpu.VMEM((K,), jnp.int32)] * NE,
        [pltpu.VMEM((K,), jnp.float32)] * NE,
        [pltpu.VMEM((K, D), jnp.float32)] * 2,
        pltpu.VMEM_SHARED((NPAD, D), jnp.float32),
        [pltpu.SemaphoreType.DMA] * 2,
        [pltpu.SemaphoreType.DMA] * 2,
        [pltpu.SemaphoreType.DMA] * NE,
        [pltpu.SemaphoreType.DMA] * NE,
        [pltpu.SemaphoreType.DMA] * NE,
    ],
)
def _sc_spmm(g_hbm, src_hbm, dst_hbm, w_hbm, out_hbm,
             srcs, dsts, ws, rows, acc_s, gsems, xsems, ssems, dsems, wsems):
    cid = lax.axis_index("c")
    sid = lax.axis_index("s")
    wid = sid * NC + cid
    r0 = rows[0]

    def _stage(e, j):
        pltpu.async_copy(src_hbm.at[wid, j], srcs[e], ssems[e])
        pltpu.async_copy(dst_hbm.at[wid, j], dsts[e], dsems[e])
        pltpu.async_copy(w_hbm.at[wid, j], ws[e], wsems[e])

    def _stage_wait(e):
        pltpu.make_async_copy(src_hbm.at[wid, 0], srcs[e], ssems[e]).wait()
        pltpu.make_async_copy(dst_hbm.at[wid, 0], dsts[e], dsems[e]).wait()
        pltpu.make_async_copy(w_hbm.at[wid, 0], ws[e], wsems[e]).wait()

    def _gather_wait(r):
        pltpu.make_async_copy(g_hbm.at[pl.ds(0, K)], rows[r], gsems[r]).wait()

    def _scale(r, e):
        def _grp(g, _):
            wvec = ws[e][pl.ds(g * 16, 16)]
            for rr in range(16):
                sc = wvec[rr]
                for i in range(D // 16):
                    sl = pl.ds(i * 16, 16)
                    rows[r][g * 16 + rr, sl] = rows[r][g * 16 + rr, sl] * sc
            return 0

        lax.fori_loop(0, K // 16, _grp, 0)

    def _zero(r, _):
        for i in range(D // 16):
            r0[r, pl.ds(i * 16, 16)] = jnp.zeros((16,), jnp.float32)
        return 0

    lax.fori_loop(0, K, _zero, 0)
    for t in range(RPT // K):
        pltpu.sync_copy(r0, acc_s.at[pl.ds(sid * RPT + t * K, K)])
    plsc.subcore_barrier()

    for e in range(NE):
        _stage(e, e)
    _stage_wait(0)
    pltpu.async_copy(g_hbm.at[srcs[0]], rows[0], gsems[0])
    _stage_wait(1)
    pltpu.async_copy(g_hbm.at[srcs[1]], rows[1], gsems[1])

    def _pair(eA, eB, jA):
        _gather_wait(0)
        _scale(0, eA)
        dA = pltpu.async_copy(rows[0], acc_s.at[dsts[eA]], xsems[0], add=True)
        _gather_wait(1)
        _scale(1, eB)
        dB = pltpu.async_copy(rows[1], acc_s.at[dsts[eB]], xsems[1], add=True)
        dA.wait()
        _stage_wait((eA + 2) % NE)
        pltpu.async_copy(g_hbm.at[srcs[(eA + 2) % NE]], rows[0], gsems[0])
        dB.wait()
        _stage_wait((eB + 2) % NE)
        pltpu.async_copy(g_hbm.at[srcs[(eB + 2) % NE]], rows[1], gsems[1])
        _stage(eA, jnp.minimum(jA + NE, C - 1))
        _stage(eB, jnp.minimum(jA + 1 + NE, C - 1))

    def _body(q, _):
        j0 = q * NE
        _pair(0, 1, j0)
        _pair(2, 3, j0 + 2)
        return 0

    lax.fori_loop(0, C // NE, _body, 0)
    _gather_wait(0)
    _gather_wait(1)
    _stage_wait(2)
    _stage_wait(3)
    plsc.subcore_barrier()

    for t in range(RPT // K):
        pltpu.sync_copy(acc_s.at[pl.ds(sid * RPT + t * K, K)], r0)
        pltpu.sync_copy(r0, out_hbm.at[cid, pl.ds(sid * RPT + t * K, K)])


def _tc_dinv_body(deg_ref, out_ref):
    gid = lax.broadcasted_iota(jnp.int32, (NPAD,), 0)
    deg = deg_ref[0] + deg_ref[1] + jnp.where(gid < N, 1.0, 0.0)
    out_ref[...] = jnp.where(deg > 0, lax.rsqrt(jnp.maximum(deg, 1e-12)), 0.0)


def _tc_embed_body(x_ref, we_ref, w1_ref, dinv_ref, g_ref):
    h0 = jnp.dot(x_ref[...], we_ref[...], preferred_element_type=jnp.float32)
    h1 = jnp.dot(h0, w1_ref[...], preferred_element_type=jnp.float32)
    g_ref[...] = dinv_ref[...] * h1


def _tc_step_body(p_ref, g_ref, dinv_ref, b_ref, w1_ref, y_ref, gn_ref):
    s = p_ref[0, :N, :] + p_ref[1, :N, :] + g_ref[...]
    y = dinv_ref[...] * s + b_ref[...]
    y_ref[...] = y
    a = jnp.maximum(y, 0.0)
    h = jnp.dot(a, w1_ref[...], preferred_element_type=jnp.float32)
    gn_ref[...] = dinv_ref[...] * h


def _tc_call(body, out_shapes, *args):
    outs = pl.pallas_call(
        body,
        out_shape=[jax.ShapeDtypeStruct(s, jnp.float32) for s in out_shapes],
    )(*args)
    return outs[0] if len(out_shapes) == 1 else outs


def kernel(x, pp_edge_index, edge_weight, W_embed, W1, b1):
    src = pp_edge_index[0]
    dst = pp_edge_index[1]

    pad = EPAD - E
    spread = (jnp.arange(pad, dtype=jnp.int32) * 97) % N
    src_p = jnp.concatenate([src, spread]).reshape(NW, C, K)
    dst_p = jnp.concatenate([dst, spread]).reshape(NW, C, K)
    w_p = jnp.concatenate(
        [edge_weight, jnp.zeros((pad,), jnp.float32)]
    ).reshape(NW, C, K)

    deg_part = _sc_deg(dst_p, w_p)
    dinv = _tc_call(_tc_dinv_body, [(NPAD,)], deg_part)
    dinv_col = dinv[:N, None]

    b_row = b1[None, :]
    g1 = _tc_call(_tc_embed_body, [(N, D)], x, W_embed, W1, dinv_col)

    p1 = _sc_spmm(g1, src_p, dst_p, w_p)
    _, g2 = _tc_call(_tc_step_body, [(N, D), (N, D)],
                     p1, g1, dinv_col, b_row, W1)
    p2 = _sc_spmm(g2, src_p, dst_p, w_p)
    out, _ = _tc_call(_tc_step_body, [(N, D), (N, D)],
                      p2, g2, dinv_col, b_row, W1)
    return out
